# baseline (speedup 1.0000x reference)
import functools

import jax
import jax.numpy as jnp
from jax.experimental import pallas as pl

_RB = 256


def _bmu_kernel(x_ref, xsq_ref, w_ref, wsq_ref, idx_ref, q_ref, *, n, chunk):
    x = x_ref[...]
    xsq = xsq_ref[...]
    run_min = jnp.full((_RB,), jnp.inf, dtype=jnp.float32)
    run_idx = jnp.zeros((_RB,), dtype=jnp.int32)
    for c in range(n // chunk):
        w = w_ref[pl.ds(c * chunk, chunk), :]
        wsq = wsq_ref[0, pl.ds(c * chunk, chunk)]
        dot = jax.lax.dot_general(
            x, w, (((1,), (1,)), ((), ())),
            preferred_element_type=jnp.float32,
        )
        d2 = (xsq + wsq[None, :]) - 2.0 * dot
        d2 = jnp.maximum(d2, 0.0)
        cmin = jnp.min(d2, axis=1)
        io = jax.lax.broadcasted_iota(jnp.int32, (_RB, chunk), 1)
        cidx = jnp.min(jnp.where(d2 == cmin[:, None], io, chunk), axis=1)
        cidx = cidx + c * chunk
        take = cmin < run_min
        run_min = jnp.where(take, cmin, run_min)
        run_idx = jnp.where(take, cidx, run_idx)
    idx_ref[...] = run_idx[:, None]
    q_ref[...] = jnp.sqrt(run_min + 1e-12)[:, None]


def _bmu_level(x, xsq, Wf, wsq, chunk):
    rows, d = x.shape
    n = Wf.shape[0]
    idx, q = pl.pallas_call(
        functools.partial(_bmu_kernel, n=n, chunk=chunk),
        grid=(rows // _RB,),
        in_specs=[
            pl.BlockSpec((_RB, d), lambda i: (i, 0)),
            pl.BlockSpec((_RB, 1), lambda i: (i, 0)),
            pl.BlockSpec((n, d), lambda i: (0, 0)),
            pl.BlockSpec((1, n), lambda i: (0, 0)),
        ],
        out_specs=[
            pl.BlockSpec((_RB, 1), lambda i: (i, 0)),
            pl.BlockSpec((_RB, 1), lambda i: (i, 0)),
        ],
        out_shape=[
            jax.ShapeDtypeStruct((rows, 1), jnp.int32),
            jax.ShapeDtypeStruct((rows, 1), jnp.float32),
        ],
    )(x, xsq, Wf, wsq)
    return idx[:, 0], q[:, 0]


def kernel(x, W1, W2, W3):
    xsq = jnp.sum(x * x, axis=1, keepdims=True)
    coords_out, qerr_out = [], []
    for W in (W1, W2, W3):
        g0, g1, d = W.shape
        n = g0 * g1
        Wf = W.reshape(n, d)
        wsq = jnp.sum(Wf * Wf, axis=1)[None, :]
        idx, q = _bmu_level(x, xsq, Wf, wsq, chunk=min(512, n))
        coords = jnp.stack([idx // g1, idx % g1], axis=1).astype(jnp.int32)
        coords_out.append(coords)
        qerr_out.append(q)
    return (coords_out[0], coords_out[1], coords_out[2],
            qerr_out[0], qerr_out[1], qerr_out[2])


# speedup vs baseline: 1.4650x; 1.4650x over previous
import functools

import jax
import jax.numpy as jnp
from jax.experimental import pallas as pl

_RB = 1024


def _bmu_kernel(x_ref, xsq_ref, w_ref, wsq_ref, io_ref, idx_ref, q_ref, *, n, chunk):
    x = x_ref[...]
    xsq = xsq_ref[...]
    run_min = jnp.full((_RB,), jnp.inf, dtype=jnp.float32)
    run_idx = jnp.zeros((_RB,), dtype=jnp.float32)
    for c in range(n // chunk):
        w = w_ref[pl.ds(c * chunk, chunk), :]
        wsq = wsq_ref[0, pl.ds(c * chunk, chunk)]
        io = io_ref[0, pl.ds(c * chunk, chunk)]
        dotneg = jax.lax.dot_general(
            x, w, (((1,), (1,)), ((), ())),
            preferred_element_type=jnp.float32,
        )
        d2 = (xsq + wsq[None, :]) + dotneg
        cmin = jnp.min(d2, axis=1)
        cidx = jnp.min(jnp.where(d2 == cmin[:, None], io[None, :], float(n)), axis=1)
        take = cmin < run_min
        run_min = jnp.where(take, cmin, run_min)
        run_idx = jnp.where(take, cidx, run_idx)
    idx_ref[...] = run_idx.astype(jnp.int32)[:, None]
    q_ref[...] = jnp.sqrt(jnp.maximum(run_min, 0.0) + 1e-12)[:, None]


def _bmu_level(x, xsq, Wneg, wsq, io, chunk):
    rows, d = x.shape
    n = Wneg.shape[0]
    idx, q = pl.pallas_call(
        functools.partial(_bmu_kernel, n=n, chunk=chunk),
        grid=(rows // _RB,),
        in_specs=[
            pl.BlockSpec((_RB, d), lambda i: (i, 0)),
            pl.BlockSpec((_RB, 1), lambda i: (i, 0)),
            pl.BlockSpec((n, d), lambda i: (0, 0)),
            pl.BlockSpec((1, n), lambda i: (0, 0)),
            pl.BlockSpec((1, n), lambda i: (0, 0)),
        ],
        out_specs=[
            pl.BlockSpec((_RB, 1), lambda i: (i, 0)),
            pl.BlockSpec((_RB, 1), lambda i: (i, 0)),
        ],
        out_shape=[
            jax.ShapeDtypeStruct((rows, 1), jnp.int32),
            jax.ShapeDtypeStruct((rows, 1), jnp.float32),
        ],
    )(x, xsq, Wneg, wsq, io)
    return idx[:, 0], q[:, 0]


def kernel(x, W1, W2, W3):
    xsq = jnp.sum(x * x, axis=1, keepdims=True)
    coords_out, qerr_out = [], []
    for W in (W1, W2, W3):
        g0, g1, d = W.shape
        n = g0 * g1
        Wf = W.reshape(n, d)
        wsq = jnp.sum(Wf * Wf, axis=1)[None, :]
        Wneg = Wf * (-2.0)
        io = jnp.arange(n, dtype=jnp.float32)[None, :]
        idx, q = _bmu_level(x, xsq, Wneg, wsq, io, chunk=min(512, n))
        coords = jnp.stack([idx // g1, idx % g1], axis=1).astype(jnp.int32)
        coords_out.append(coords)
        qerr_out.append(q)
    return (coords_out[0], coords_out[1], coords_out[2],
            qerr_out[0], qerr_out[1], qerr_out[2])


# speedup vs baseline: 1.6223x; 1.1074x over previous
import functools

import jax
import jax.numpy as jnp
from jax.experimental import pallas as pl

_RB = 1024
_CHUNK = 512


def _level_scan(x, xsq, w_ref, wsq_ref, io_ref, idx_ref, q_ref, n):
    chunk = min(_CHUNK, n)
    run_min = jnp.full((_RB,), jnp.inf, dtype=jnp.float32)
    run_idx = jnp.zeros((_RB,), dtype=jnp.float32)
    for c in range(n // chunk):
        w = w_ref[pl.ds(c * chunk, chunk), :]
        wsq = wsq_ref[0, pl.ds(c * chunk, chunk)]
        io = io_ref[0, pl.ds(c * chunk, chunk)]
        dotneg = jax.lax.dot_general(
            x, w, (((1,), (1,)), ((), ())),
            preferred_element_type=jnp.float32,
        )
        d2 = (xsq + wsq[None, :]) + dotneg
        cmin = jnp.min(d2, axis=1)
        cidx = jnp.min(jnp.where(d2 == cmin[:, None], io[None, :], float(n)), axis=1)
        take = cmin < run_min
        run_min = jnp.where(take, cmin, run_min)
        run_idx = jnp.where(take, cidx, run_idx)
    idx_ref[...] = run_idx.astype(jnp.int32)[:, None]
    q_ref[...] = jnp.sqrt(jnp.maximum(run_min, 0.0) + 1e-12)[:, None]


def _bmu_kernel(x_ref, xsq_ref,
                w1_ref, wsq1_ref, io1_ref,
                w2_ref, wsq2_ref, io2_ref,
                w3_ref, wsq3_ref, io3_ref,
                idx1_ref, q1_ref, idx2_ref, q2_ref, idx3_ref, q3_ref,
                *, n1, n2, n3):
    x = x_ref[...]
    xsq = xsq_ref[...]
    _level_scan(x, xsq, w1_ref, wsq1_ref, io1_ref, idx1_ref, q1_ref, n1)
    _level_scan(x, xsq, w2_ref, wsq2_ref, io2_ref, idx2_ref, q2_ref, n2)
    _level_scan(x, xsq, w3_ref, wsq3_ref, io3_ref, idx3_ref, q3_ref, n3)


def _row_spec(d):
    return pl.BlockSpec((_RB, d), lambda i: (i, 0))


def _full_spec(shape):
    return pl.BlockSpec(shape, lambda i: (0, 0))


def kernel(x, W1, W2, W3):
    rows, d = x.shape
    xsq = jnp.sum(x * x, axis=1, keepdims=True)
    args = [x, xsq]
    in_specs = [_row_spec(d), _row_spec(1)]
    ns = []
    for W in (W1, W2, W3):
        g0, g1, _ = W.shape
        n = g0 * g1
        ns.append(n)
        Wf = W.reshape(n, d)
        wsq = jnp.sum(Wf * Wf, axis=1)[None, :]
        args += [Wf * (-2.0), wsq, jnp.arange(n, dtype=jnp.float32)[None, :]]
        in_specs += [_full_spec((n, d)), _full_spec((1, n)), _full_spec((1, n))]
    out_specs = [_row_spec(1)] * 6
    out_shape = []
    for _ in range(3):
        out_shape += [jax.ShapeDtypeStruct((rows, 1), jnp.int32),
                      jax.ShapeDtypeStruct((rows, 1), jnp.float32)]
    idx1, q1, idx2, q2, idx3, q3 = pl.pallas_call(
        functools.partial(_bmu_kernel, n1=ns[0], n2=ns[1], n3=ns[2]),
        grid=(rows // _RB,),
        in_specs=in_specs,
        out_specs=out_specs,
        out_shape=out_shape,
    )(*args)
    coords = []
    for idx, W in ((idx1, W1), (idx2, W2), (idx3, W3)):
        g1 = W.shape[1]
        flat = idx[:, 0]
        coords.append(jnp.stack([flat // g1, flat % g1], axis=1).astype(jnp.int32))
    return (coords[0], coords[1], coords[2], q1[:, 0], q2[:, 0], q3[:, 0])


# speedup vs baseline: 2.0340x; 1.2537x over previous
import functools

import jax
import jax.numpy as jnp
from jax.experimental import pallas as pl

_RB = 1024
_CHUNK = 512


def _level_scan(x2, xsq, w_ref, wsq_ref, io_ref, coord_ref, q_ref, n, g1):
    chunk = min(_CHUNK, n)
    run_min = jnp.full((_RB,), jnp.inf, dtype=jnp.float32)
    run_idx = jnp.zeros((_RB,), dtype=jnp.float32)
    for c in range(n // chunk):
        w = w_ref[pl.ds(c * chunk, chunk), :]
        wsq = wsq_ref[0, pl.ds(c * chunk, chunk)]
        io = io_ref[0, pl.ds(c * chunk, chunk)]
        dotneg = jax.lax.dot_general(
            x2, w, (((1,), (1,)), ((), ())),
            preferred_element_type=jnp.float32,
        )
        d2 = (xsq + wsq[None, :]) + dotneg
        cmin = jnp.min(d2, axis=1)
        cidx = jnp.min(jnp.where(d2 == cmin[:, None], io[None, :], float(n)), axis=1)
        take = cmin < run_min
        run_min = jnp.where(take, cmin, run_min)
        run_idx = jnp.where(take, cidx, run_idx)
    row = jnp.floor(run_idx * (1.0 / g1))
    col = run_idx - row * g1
    coord_ref[...] = jnp.concatenate(
        [row[:, None], col[:, None]], axis=1).astype(jnp.int32)
    q_ref[...] = jnp.sqrt(jnp.maximum(run_min, 0.0) + 1e-12)[:, None]


def _bmu_kernel(x_ref, xsq_ref,
                w1_ref, wsq1_ref, io1_ref,
                w2_ref, wsq2_ref, io2_ref,
                w3_ref, wsq3_ref, io3_ref,
                c1_ref, q1_ref, c2_ref, q2_ref, c3_ref, q3_ref,
                *, n1, n2, n3, g1s):
    x2 = x_ref[...] * (-2.0)
    xsq = xsq_ref[...]
    _level_scan(x2, xsq, w1_ref, wsq1_ref, io1_ref, c1_ref, q1_ref, n1, g1s[0])
    _level_scan(x2, xsq, w2_ref, wsq2_ref, io2_ref, c2_ref, q2_ref, n2, g1s[1])
    _level_scan(x2, xsq, w3_ref, wsq3_ref, io3_ref, c3_ref, q3_ref, n3, g1s[2])


def _row_spec(d):
    return pl.BlockSpec((_RB, d), lambda i: (i, 0))


def _full_spec(shape):
    return pl.BlockSpec(shape, lambda i: (0, 0))


def kernel(x, W1, W2, W3):
    rows, d = x.shape
    xsq = jnp.sum(x * x, axis=1, keepdims=True)
    args = [x, xsq]
    in_specs = [_row_spec(d), _row_spec(1)]
    ns, g1s = [], []
    for W in (W1, W2, W3):
        g0, g1, _ = W.shape
        n = g0 * g1
        ns.append(n)
        g1s.append(float(g1))
        Wf = W.reshape(n, d)
        wsq = jnp.sum(Wf * Wf, axis=1)[None, :]
        args += [Wf, wsq, jnp.arange(n, dtype=jnp.float32)[None, :]]
        in_specs += [_full_spec((n, d)), _full_spec((1, n)), _full_spec((1, n))]
    out_specs = [_row_spec(2), _row_spec(1)] * 3
    out_shape = []
    for _ in range(3):
        out_shape += [jax.ShapeDtypeStruct((rows, 2), jnp.int32),
                      jax.ShapeDtypeStruct((rows, 1), jnp.float32)]
    c1, q1, c2, q2, c3, q3 = pl.pallas_call(
        functools.partial(_bmu_kernel, n1=ns[0], n2=ns[1], n3=ns[2],
                          g1s=tuple(g1s)),
        grid=(rows // _RB,),
        in_specs=in_specs,
        out_specs=out_specs,
        out_shape=out_shape,
    )(*args)
    return (c1, c2, c3, q1[:, 0], q2[:, 0], q3[:, 0])


# speedup vs baseline: 2.0999x; 1.0324x over previous
import functools

import jax
import jax.numpy as jnp
from jax.experimental import pallas as pl

_RB = 1024
_CHUNK = 512


def _level_scan(x2, xsq, w_ref, wsq_ref, coord_ref, q_ref, off, n, g1):
    chunk = min(_CHUNK, n)
    run_min = jnp.full((_RB,), jnp.inf, dtype=jnp.float32)
    run_idx = jnp.zeros((_RB,), dtype=jnp.float32)
    io = jax.lax.broadcasted_iota(jnp.int32, (1, chunk), 1).astype(jnp.float32)
    for c in range(n // chunk):
        w = w_ref[pl.ds(off + c * chunk, chunk), :]
        wsq = wsq_ref[0, pl.ds(off + c * chunk, chunk)]
        dotneg = jax.lax.dot_general(
            x2, w, (((1,), (1,)), ((), ())),
            preferred_element_type=jnp.float32,
        )
        d2 = (xsq + wsq[None, :]) + dotneg
        cmin = jnp.min(d2, axis=1)
        cidx = jnp.min(jnp.where(d2 == cmin[:, None], io, float(chunk)), axis=1)
        take = cmin < run_min
        run_min = jnp.where(take, cmin, run_min)
        run_idx = jnp.where(take, cidx + float(c * chunk), run_idx)
    row = jnp.floor(run_idx * (1.0 / g1))
    col = run_idx - row * g1
    coord_ref[...] = jnp.concatenate(
        [row[:, None], col[:, None]], axis=1).astype(jnp.int32)
    q_ref[...] = jnp.sqrt(jnp.maximum(run_min, 0.0) + 1e-12)[:, None]


def _bmu_kernel(x_ref, xsq_ref, w_ref, wsq_ref,
                c1_ref, q1_ref, c2_ref, q2_ref, c3_ref, q3_ref,
                *, ns, g1s):
    x = x_ref[...]
    xsq = xsq_ref[...]
    x2 = x * (-2.0)
    outs = ((c1_ref, q1_ref), (c2_ref, q2_ref), (c3_ref, q3_ref))
    off = 0
    for (c_ref, q_ref), n, g1 in zip(outs, ns, g1s):
        _level_scan(x2, xsq, w_ref, wsq_ref, c_ref, q_ref, off, n, g1)
        off += n


def _row_spec(d):
    return pl.BlockSpec((_RB, d), lambda i: (i, 0))


def _full_spec(shape):
    return pl.BlockSpec(shape, lambda i: (0, 0))


def kernel(x, W1, W2, W3):
    rows, d = x.shape
    xsq = jnp.sum(x * x, axis=1, keepdims=True)
    ns, g1s = [], []
    flats = []
    for W in (W1, W2, W3):
        g0, g1, _ = W.shape
        ns.append(g0 * g1)
        g1s.append(float(g1))
        flats.append(W.reshape(g0 * g1, d))
    w_all = jnp.concatenate(flats, axis=0)
    wsq_all = jnp.sum(w_all * w_all, axis=1)[None, :]
    n_tot = w_all.shape[0]
    out_specs = [_row_spec(2), _row_spec(1)] * 3
    out_shape = []
    for _ in range(3):
        out_shape += [jax.ShapeDtypeStruct((rows, 2), jnp.int32),
                      jax.ShapeDtypeStruct((rows, 1), jnp.float32)]
    c1, q1, c2, q2, c3, q3 = pl.pallas_call(
        functools.partial(_bmu_kernel, ns=tuple(ns), g1s=tuple(g1s)),
        grid=(rows // _RB,),
        in_specs=[_row_spec(d), _row_spec(1),
                  _full_spec((n_tot, d)), _full_spec((1, n_tot))],
        out_specs=out_specs,
        out_shape=out_shape,
    )(x, xsq, w_all, wsq_all)
    return (c1, c2, c3, q1[:, 0], q2[:, 0], q3[:, 0])


# speedup vs baseline: 2.3054x; 1.0979x over previous
import functools

import jax
import jax.numpy as jnp
from jax.experimental import pallas as pl
from jax.experimental.pallas import tpu as pltpu

_RB = 1024
_CHUNK = 512


def _level_scan(x2, xsq, w_ref, wsq_ref, coord_ref, q_ref, n, g1):
    chunk = min(_CHUNK, n)
    run_min = jnp.full((_RB,), jnp.inf, dtype=jnp.float32)
    run_idx = jnp.zeros((_RB,), dtype=jnp.float32)
    io = jax.lax.broadcasted_iota(jnp.int32, (1, chunk), 1).astype(jnp.float32)
    for c in range(n // chunk):
        w = w_ref[pl.ds(c * chunk, chunk), :]
        wsq = wsq_ref[0, pl.ds(c * chunk, chunk)]
        dotneg = jax.lax.dot_general(
            x2, w, (((1,), (1,)), ((), ())),
            preferred_element_type=jnp.float32,
        )
        d2 = (xsq + wsq[None, :]) + dotneg
        cmin = jnp.min(d2, axis=1)
        cidx = jnp.min(jnp.where(d2 == cmin[:, None], io, float(chunk)), axis=1)
        take = cmin < run_min
        run_min = jnp.where(take, cmin, run_min)
        run_idx = jnp.where(take, cidx + float(c * chunk), run_idx)
    row = jnp.floor(run_idx * (1.0 / g1))
    col = run_idx - row * g1
    coord_ref[...] = jnp.concatenate(
        [row[:, None], col[:, None]], axis=1).astype(jnp.int32)
    q_ref[...] = jnp.sqrt(jnp.maximum(run_min, 0.0) + 1e-12)[:, None]


def _bmu_kernel(x_ref,
                w1_ref, w2_ref, w3_ref,
                c1_ref, q1_ref, c2_ref, q2_ref, c3_ref, q3_ref,
                wsq1_ref, wsq2_ref, wsq3_ref,
                *, ns, g1s):
    @pl.when(pl.program_id(0) == 0)
    def _():
        for w_ref, wsq_ref in ((w1_ref, wsq1_ref), (w2_ref, wsq2_ref),
                               (w3_ref, wsq3_ref)):
            w = w_ref[...]
            s = jnp.sum(w * w, axis=1)
            wsq_ref[...] = s[None, :]

    x = x_ref[...]
    xsq = jnp.sum(x * x, axis=1, keepdims=True)
    x2 = x * (-2.0)
    scans = ((w1_ref, wsq1_ref, c1_ref, q1_ref),
             (w2_ref, wsq2_ref, c2_ref, q2_ref),
             (w3_ref, wsq3_ref, c3_ref, q3_ref))
    for (w_ref, wsq_ref, c_ref, q_ref), n, g1 in zip(scans, ns, g1s):
        _level_scan(x2, xsq, w_ref, wsq_ref, c_ref, q_ref, n, g1)


def _row_spec(d):
    return pl.BlockSpec((_RB, d), lambda i: (i, 0))


def _full_spec(shape):
    return pl.BlockSpec(shape, lambda i: (0, 0))


def kernel(x, W1, W2, W3):
    rows, d = x.shape
    args = [x]
    in_specs = [_row_spec(d)]
    ns, g1s = [], []
    for W in (W1, W2, W3):
        g0, g1, _ = W.shape
        ns.append(g0 * g1)
        g1s.append(float(g1))
        args.append(W.reshape(g0 * g1, d))
        in_specs.append(_full_spec((g0 * g1, d)))
    out_specs = [_row_spec(2), _row_spec(1)] * 3
    out_shape = []
    for _ in range(3):
        out_shape += [jax.ShapeDtypeStruct((rows, 2), jnp.int32),
                      jax.ShapeDtypeStruct((rows, 1), jnp.float32)]
    c1, q1, c2, q2, c3, q3 = pl.pallas_call(
        functools.partial(_bmu_kernel, ns=tuple(ns), g1s=tuple(g1s)),
        grid=(rows // _RB,),
        in_specs=in_specs,
        out_specs=out_specs,
        out_shape=out_shape,
        scratch_shapes=[pltpu.VMEM((1, ns[0]), jnp.float32),
                        pltpu.VMEM((1, ns[1]), jnp.float32),
                        pltpu.VMEM((1, ns[2]), jnp.float32)],
    )(*args)
    return (c1, c2, c3, q1[:, 0], q2[:, 0], q3[:, 0])
